# baseline (speedup 1.0000x reference)
import functools

import jax
import jax.numpy as jnp
from jax import lax
from jax.experimental import pallas as pl
from jax.experimental.pallas import tpu as pltpu
from jax.experimental.pallas import tpu_sc as plsc

_N = 10000
_E = 320000
_D = 128
_LAYERS = 3
_MLP_H = 32
_CLS = 10

_NC = 2
_NS = 16
_NW = _NC * _NS
_CH = 128
_NCHUNK = 2 * (-(-_E // (_NW * _CH * 2)))
_EPAD = _NW * _NCHUNK * _CH
_NPAD = 10240
_RPT = _NPAD // _NS


def _sc_propagate_body(m_hbm, src_hbm, dst_hbm, attr_hbm, zero_hbm, out_hbm,
                       src_v, dst_v, attr_v, rows_v, sem, acc_sh):
    c = lax.axis_index("c")
    s = lax.axis_index("s")
    w = c * _NS + s

    pltpu.sync_copy(zero_hbm, acc_sh.at[pl.ds(s * _RPT, _RPT)])
    plsc.subcore_barrier()

    def chunk_body(j, carry):
        pltpu.sync_copy(src_hbm.at[w, j], src_v)
        pltpu.sync_copy(dst_hbm.at[w, j], dst_v)
        pltpu.sync_copy(attr_hbm.at[w, j], attr_v)
        pltpu.async_copy(m_hbm.at[src_v], rows_v, sem).wait()

        def group_body(g, carry2):
            av = attr_v[pl.ds(g * 16, 16)]
            base = g * 16
            for jj in range(16):
                a = av[jj]
                for cc in range(_D // 16):
                    sl = pl.ds(cc * 16, 16)
                    rows_v[base + jj, sl] = rows_v[base + jj, sl] * a
            return carry2

        lax.fori_loop(0, _CH // 16, group_body, 0)
        pltpu.sync_copy(rows_v, acc_sh.at[dst_v], add=True)
        return carry

    lax.fori_loop(0, _NCHUNK, chunk_body, 0)
    plsc.subcore_barrier()
    pltpu.sync_copy(acc_sh.at[pl.ds(s * _RPT, _RPT)],
                    out_hbm.at[c, pl.ds(s * _RPT, _RPT)])


@functools.cache
def _sc_propagate_kernel():
    return pl.kernel(
        _sc_propagate_body,
        out_type=jax.ShapeDtypeStruct((_NC, _NPAD, _D), jnp.float32),
        mesh=plsc.VectorSubcoreMesh(core_axis_name="c", subcore_axis_name="s",
                                    num_cores=_NC, num_subcores=_NS),
        scratch_types=[
            pltpu.VMEM((_CH,), jnp.int32),
            pltpu.VMEM((_CH,), jnp.int32),
            pltpu.VMEM((_CH,), jnp.float32),
            pltpu.VMEM((_CH, _D), jnp.float32),
            pltpu.SemaphoreType.DMA,
            pltpu.VMEM_SHARED((_NPAD, _D), jnp.float32),
        ],
    )


def _sc_propagate(m, src_p, dst_p, attr_p, zero_rows):
    parts = _sc_propagate_kernel()(m, src_p, dst_p, attr_p, zero_rows)
    return parts[:, :_N, :]


_BLK = 1000


def _mm_body(h_ref, w_ref, o_ref):
    o_ref[...] = jnp.dot(h_ref[...], w_ref[...],
                         preferred_element_type=jnp.float32)


def _tc_matmul(h, w):
    return pl.pallas_call(
        _mm_body,
        grid=(_N // _BLK,),
        in_specs=[
            pl.BlockSpec((_BLK, _D), lambda i: (i, 0)),
            pl.BlockSpec((_D, _D), lambda i: (0, 0)),
        ],
        out_specs=pl.BlockSpec((_BLK, _D), lambda i: (i, 0)),
        out_shape=jax.ShapeDtypeStruct((_N, _D), jnp.float32),
    )(h, w)


def _gru_body(p0_ref, p1_ref, h_ref, wih_ref, whh_ref, bih_ref, bhh_ref,
              o_ref):
    agg = p0_ref[...] + p1_ref[...]
    h = h_ref[...]
    gi = lax.dot_general(agg, wih_ref[...], (((1,), (1,)), ((), ())),
                         preferred_element_type=jnp.float32) + bih_ref[...]
    gh = lax.dot_general(h, whh_ref[...], (((1,), (1,)), ((), ())),
                         preferred_element_type=jnp.float32) + bhh_ref[...]
    r = jax.nn.sigmoid(gi[:, :_D] + gh[:, :_D])
    z = jax.nn.sigmoid(gi[:, _D:2 * _D] + gh[:, _D:2 * _D])
    n = jnp.tanh(gi[:, 2 * _D:] + r * gh[:, 2 * _D:])
    o_ref[...] = (1.0 - z) * n + z * h


def _tc_gru(p0, p1, h, w_ih, w_hh, b_ih, b_hh):
    return pl.pallas_call(
        _gru_body,
        grid=(_N // _BLK,),
        in_specs=[
            pl.BlockSpec((_BLK, _D), lambda i: (i, 0)),
            pl.BlockSpec((_BLK, _D), lambda i: (i, 0)),
            pl.BlockSpec((_BLK, _D), lambda i: (i, 0)),
            pl.BlockSpec((3 * _D, _D), lambda i: (0, 0)),
            pl.BlockSpec((3 * _D, _D), lambda i: (0, 0)),
            pl.BlockSpec((1, 3 * _D), lambda i: (0, 0)),
            pl.BlockSpec((1, 3 * _D), lambda i: (0, 0)),
        ],
        out_specs=pl.BlockSpec((_BLK, _D), lambda i: (i, 0)),
        out_shape=jax.ShapeDtypeStruct((_N, _D), jnp.float32),
    )(p0, p1, h, w_ih, w_hh, b_ih, b_hh)


def _mlp_body(h_ref, w0_ref, b0_ref, w1_ref, b1_ref, w2_ref, b2_ref,
              ow_ref, ob_ref, emb_ref, lsm_ref):
    y = jnp.tanh(lax.dot_general(h_ref[...], w0_ref[...],
                                 (((1,), (1,)), ((), ())),
                                 preferred_element_type=jnp.float32)
                 + b0_ref[...])
    y = jnp.tanh(lax.dot_general(y, w1_ref[...], (((1,), (1,)), ((), ())),
                                 preferred_element_type=jnp.float32)
                 + b1_ref[...])
    y = jnp.tanh(lax.dot_general(y, w2_ref[...], (((1,), (1,)), ((), ())),
                                 preferred_element_type=jnp.float32)
                 + b2_ref[...])
    e = lax.dot_general(y, ow_ref[...], (((1,), (1,)), ((), ())),
                        preferred_element_type=jnp.float32) + ob_ref[...]
    emb_ref[...] = e
    shifted = e - jnp.max(e, axis=-1, keepdims=True)
    lsm_ref[...] = shifted - jnp.log(
        jnp.sum(jnp.exp(shifted), axis=-1, keepdims=True))


def _tc_mlp(h, w0, b0, w1, b1, w2, b2, ow, ob):
    return pl.pallas_call(
        _mlp_body,
        grid=(_N // _BLK,),
        in_specs=[
            pl.BlockSpec((_BLK, _D), lambda i: (i, 0)),
            pl.BlockSpec((_MLP_H, _D), lambda i: (0, 0)),
            pl.BlockSpec((1, _MLP_H), lambda i: (0, 0)),
            pl.BlockSpec((_MLP_H, _MLP_H), lambda i: (0, 0)),
            pl.BlockSpec((1, _MLP_H), lambda i: (0, 0)),
            pl.BlockSpec((_MLP_H, _MLP_H), lambda i: (0, 0)),
            pl.BlockSpec((1, _MLP_H), lambda i: (0, 0)),
            pl.BlockSpec((_CLS, _MLP_H), lambda i: (0, 0)),
            pl.BlockSpec((1, _CLS), lambda i: (0, 0)),
        ],
        out_specs=[
            pl.BlockSpec((_BLK, _CLS), lambda i: (i, 0)),
            pl.BlockSpec((_BLK, _CLS), lambda i: (i, 0)),
        ],
        out_shape=[
            jax.ShapeDtypeStruct((_N, _CLS), jnp.float32),
            jax.ShapeDtypeStruct((_N, _CLS), jnp.float32),
        ],
    )(h, w0, b0, w1, b1, w2, b2, ow, ob)


def kernel(x, edge_index, edge_attr, W, W_ih, W_hh, b_ih, b_hh,
           mlp_w0, mlp_b0, mlp_w1, mlp_b1, mlp_w2, mlp_b2, out_w, out_b):
    src = edge_index[0].astype(jnp.int32)
    dst = edge_index[1].astype(jnp.int32)
    attr = edge_attr.astype(jnp.float32)

    pad = _EPAD - _E
    pad_i = jnp.zeros((pad,), jnp.int32)
    pad_d = _N + (jnp.arange(pad, dtype=jnp.int32) % (_NPAD - _N))
    src_p = jnp.concatenate([src, pad_i]).reshape(_NW, _NCHUNK, _CH)
    dst_p = jnp.concatenate([dst, pad_d]).reshape(_NW, _NCHUNK, _CH)
    attr_p = jnp.concatenate([attr, jnp.zeros((pad,), jnp.float32)]
                             ).reshape(_NW, _NCHUNK, _CH)
    zero_rows = jnp.zeros((_RPT, _D), jnp.float32)

    b_ih2 = b_ih.reshape(1, 3 * _D)
    b_hh2 = b_hh.reshape(1, 3 * _D)

    h = x
    m = _tc_matmul(h, W[0])
    for i in range(_LAYERS):
        parts = _sc_propagate(m, src_p, dst_p, attr_p, zero_rows)
        h = _tc_gru(parts[0], parts[1], h, W_ih, W_hh, b_ih2, b_hh2)
        if i + 1 < _LAYERS:
            m = _tc_matmul(h, W[i + 1])

    return _tc_mlp(h, mlp_w0, mlp_b0.reshape(1, _MLP_H),
                   mlp_w1, mlp_b1.reshape(1, _MLP_H),
                   mlp_w2, mlp_b2.reshape(1, _MLP_H),
                   out_w, out_b.reshape(1, _CLS))


# speedup vs baseline: 1.0003x; 1.0003x over previous
import functools

import jax
import jax.numpy as jnp
from jax import lax
from jax.experimental import pallas as pl
from jax.experimental.pallas import tpu as pltpu
from jax.experimental.pallas import tpu_sc as plsc

_N = 10000
_E = 320000
_D = 128
_LAYERS = 3
_MLP_H = 32
_CLS = 10

_NC = 2
_NS = 16
_NW = _NC * _NS
_CH = 128
_NCHUNK = 2 * (-(-_E // (_NW * _CH * 2)))
_EPAD = _NW * _NCHUNK * _CH
_NPAD = 10240
_RPT = _NPAD // _NS


def _sc_propagate_body(m_hbm, src_hbm, dst_hbm, attr_hbm, zero_hbm, out_hbm,
                       src_v, dst_v, attr_v, rows_v, sem, acc_sh):
    c = lax.axis_index("c")
    s = lax.axis_index("s")
    w = c * _NS + s

    pltpu.sync_copy(zero_hbm, acc_sh.at[pl.ds(s * _RPT, _RPT)])
    plsc.subcore_barrier()

    def chunk_body(j, carry):
        pltpu.sync_copy(src_hbm.at[w, j], src_v)
        pltpu.sync_copy(dst_hbm.at[w, j], dst_v)
        pltpu.sync_copy(attr_hbm.at[w, j], attr_v)
        pltpu.async_copy(m_hbm.at[src_v], rows_v, sem).wait()

        def group_body(g, carry2):
            av = attr_v[pl.ds(g * 16, 16)]
            base = g * 16
            for jj in range(16):
                a = av[jj]
                for cc in range(_D // 16):
                    sl = pl.ds(cc * 16, 16)
                    rows_v[base + jj, sl] = rows_v[base + jj, sl] * a
            return carry2

        lax.fori_loop(0, _CH // 16, group_body, 0)
        pltpu.sync_copy(rows_v, acc_sh.at[dst_v], add=True)
        return carry

    lax.fori_loop(0, _NCHUNK, chunk_body, 0)
    plsc.subcore_barrier()
    pltpu.sync_copy(acc_sh.at[pl.ds(s * _RPT, _RPT)],
                    out_hbm.at[c, pl.ds(s * _RPT, _RPT)])


@functools.cache
def _sc_propagate_kernel():
    return pl.kernel(
        _sc_propagate_body,
        out_type=jax.ShapeDtypeStruct((_NC, _NPAD, _D), jnp.float32),
        mesh=plsc.VectorSubcoreMesh(core_axis_name="c", subcore_axis_name="s",
                                    num_cores=_NC, num_subcores=_NS),
        scratch_types=[
            pltpu.VMEM((_CH,), jnp.int32),
            pltpu.VMEM((_CH,), jnp.int32),
            pltpu.VMEM((_CH,), jnp.float32),
            pltpu.VMEM((_CH, _D), jnp.float32),
            pltpu.SemaphoreType.DMA,
            pltpu.VMEM_SHARED((_NPAD, _D), jnp.float32),
        ],
    )


def _sc_propagate(m, src_p, dst_p, attr_p, zero_rows):
    parts = _sc_propagate_kernel()(m, src_p, dst_p, attr_p, zero_rows)
    return parts[:, :_N, :]


_BLK = 1000


def _mm_body(h_ref, w_ref, o_ref):
    o_ref[...] = jnp.dot(h_ref[...], w_ref[...],
                         preferred_element_type=jnp.float32)


def _tc_matmul(h, w):
    return pl.pallas_call(
        _mm_body,
        grid=(_N // _BLK,),
        in_specs=[
            pl.BlockSpec((_BLK, _D), lambda i: (i, 0)),
            pl.BlockSpec((_D, _D), lambda i: (0, 0)),
        ],
        out_specs=pl.BlockSpec((_BLK, _D), lambda i: (i, 0)),
        out_shape=jax.ShapeDtypeStruct((_N, _D), jnp.float32),
    )(h, w)


def _gru_body(p0_ref, p1_ref, h_ref, wih_ref, whh_ref, bih_ref, bhh_ref,
              o_ref):
    agg = p0_ref[...] + p1_ref[...]
    h = h_ref[...]
    gi = lax.dot_general(agg, wih_ref[...], (((1,), (1,)), ((), ())),
                         preferred_element_type=jnp.float32) + bih_ref[...]
    gh = lax.dot_general(h, whh_ref[...], (((1,), (1,)), ((), ())),
                         preferred_element_type=jnp.float32) + bhh_ref[...]
    r = jax.nn.sigmoid(gi[:, :_D] + gh[:, :_D])
    z = jax.nn.sigmoid(gi[:, _D:2 * _D] + gh[:, _D:2 * _D])
    n = jnp.tanh(gi[:, 2 * _D:] + r * gh[:, 2 * _D:])
    o_ref[...] = (1.0 - z) * n + z * h


def _tc_gru(p0, p1, h, w_ih, w_hh, b_ih, b_hh):
    return pl.pallas_call(
        _gru_body,
        grid=(_N // _BLK,),
        in_specs=[
            pl.BlockSpec((_BLK, _D), lambda i: (i, 0)),
            pl.BlockSpec((_BLK, _D), lambda i: (i, 0)),
            pl.BlockSpec((_BLK, _D), lambda i: (i, 0)),
            pl.BlockSpec((3 * _D, _D), lambda i: (0, 0)),
            pl.BlockSpec((3 * _D, _D), lambda i: (0, 0)),
            pl.BlockSpec((1, 3 * _D), lambda i: (0, 0)),
            pl.BlockSpec((1, 3 * _D), lambda i: (0, 0)),
        ],
        out_specs=pl.BlockSpec((_BLK, _D), lambda i: (i, 0)),
        out_shape=jax.ShapeDtypeStruct((_N, _D), jnp.float32),
    )(p0, p1, h, w_ih, w_hh, b_ih, b_hh)


def _mlp_body(h_ref, w0_ref, b0_ref, w1_ref, b1_ref, w2_ref, b2_ref,
              ow_ref, ob_ref, emb_ref, lsm_ref):
    y = jnp.tanh(lax.dot_general(h_ref[...], w0_ref[...],
                                 (((1,), (1,)), ((), ())),
                                 preferred_element_type=jnp.float32)
                 + b0_ref[...])
    y = jnp.tanh(lax.dot_general(y, w1_ref[...], (((1,), (1,)), ((), ())),
                                 preferred_element_type=jnp.float32)
                 + b1_ref[...])
    y = jnp.tanh(lax.dot_general(y, w2_ref[...], (((1,), (1,)), ((), ())),
                                 preferred_element_type=jnp.float32)
                 + b2_ref[...])
    e = lax.dot_general(y, ow_ref[...], (((1,), (1,)), ((), ())),
                        preferred_element_type=jnp.float32) + ob_ref[...]
    emb_ref[...] = e
    shifted = e - jnp.max(e, axis=-1, keepdims=True)
    lsm_ref[...] = shifted - jnp.log(
        jnp.sum(jnp.exp(shifted), axis=-1, keepdims=True))


def _tc_mlp(h, w0, b0, w1, b1, w2, b2, ow, ob):
    return pl.pallas_call(
        _mlp_body,
        grid=(_N // _BLK,),
        in_specs=[
            pl.BlockSpec((_BLK, _D), lambda i: (i, 0)),
            pl.BlockSpec((_MLP_H, _D), lambda i: (0, 0)),
            pl.BlockSpec((1, _MLP_H), lambda i: (0, 0)),
            pl.BlockSpec((_MLP_H, _MLP_H), lambda i: (0, 0)),
            pl.BlockSpec((1, _MLP_H), lambda i: (0, 0)),
            pl.BlockSpec((_MLP_H, _MLP_H), lambda i: (0, 0)),
            pl.BlockSpec((1, _MLP_H), lambda i: (0, 0)),
            pl.BlockSpec((_CLS, _MLP_H), lambda i: (0, 0)),
            pl.BlockSpec((1, _CLS), lambda i: (0, 0)),
        ],
        out_specs=[
            pl.BlockSpec((_BLK, _CLS), lambda i: (i, 0)),
            pl.BlockSpec((_BLK, _CLS), lambda i: (i, 0)),
        ],
        out_shape=[
            jax.ShapeDtypeStruct((_N, _CLS), jnp.float32),
            jax.ShapeDtypeStruct((_N, _CLS), jnp.float32),
        ],
    )(h, w0, b0, w1, b1, w2, b2, ow, ob)


def kernel(x, edge_index, edge_attr, W, W_ih, W_hh, b_ih, b_hh,
           mlp_w0, mlp_b0, mlp_w1, mlp_b1, mlp_w2, mlp_b2, out_w, out_b):
    src = edge_index[0].astype(jnp.int32)
    dst = edge_index[1].astype(jnp.int32)
    attr = edge_attr.astype(jnp.float32)

    pad = _EPAD - _E
    pad_i = jnp.zeros((pad,), jnp.int32)
    src_p = jnp.concatenate([src, pad_i]).reshape(_NW, _NCHUNK, _CH)
    dst_p = jnp.concatenate([dst, pad_i]).reshape(_NW, _NCHUNK, _CH)
    attr_p = jnp.concatenate([attr, jnp.zeros((pad,), jnp.float32)]
                             ).reshape(_NW, _NCHUNK, _CH)
    zero_rows = jnp.zeros((_RPT, _D), jnp.float32)

    b_ih2 = b_ih.reshape(1, 3 * _D)
    b_hh2 = b_hh.reshape(1, 3 * _D)

    h = x
    m = _tc_matmul(h, W[0])
    for i in range(_LAYERS):
        parts = _sc_propagate(m, src_p, dst_p, attr_p, zero_rows)
        h = _tc_gru(parts[0], parts[1], h, W_ih, W_hh, b_ih2, b_hh2)
        if i + 1 < _LAYERS:
            m = _tc_matmul(h, W[i + 1])

    return _tc_mlp(h, mlp_w0, mlp_b0.reshape(1, _MLP_H),
                   mlp_w1, mlp_b1.reshape(1, _MLP_H),
                   mlp_w2, mlp_b2.reshape(1, _MLP_H),
                   out_w, out_b.reshape(1, _CLS))


# speedup vs baseline: 1.3371x; 1.3367x over previous
import functools

import jax
import jax.numpy as jnp
from jax import lax
from jax.experimental import pallas as pl
from jax.experimental.pallas import tpu as pltpu
from jax.experimental.pallas import tpu_sc as plsc

_N = 10000
_E = 320000
_D = 128
_LAYERS = 3
_MLP_H = 32
_CLS = 10

_NC = 2
_NS = 16
_NW = _NC * _NS
_CH = 128
_NCHUNK = -(-_E // (_NW * _CH))
_EPAD = _NW * _NCHUNK * _CH
_NPAD = 10240
_RPT = _NPAD // _NS


def _sc_propagate_body(m_hbm, src_hbm, dst_hbm, attr_hbm, zero_hbm, out_hbm,
                       src_v, dst_v, attr_v, rows_v, sem, acc_sh):
    c = lax.axis_index("c")
    s = lax.axis_index("s")
    w = c * _NS + s

    pltpu.sync_copy(zero_hbm, acc_sh.at[pl.ds(s * _RPT, _RPT)])
    plsc.subcore_barrier()

    def chunk_body(j, carry):
        pltpu.sync_copy(src_hbm.at[w, j], src_v)
        pltpu.sync_copy(dst_hbm.at[w, j], dst_v)
        pltpu.sync_copy(attr_hbm.at[w, j], attr_v)
        pltpu.async_copy(m_hbm.at[src_v], rows_v, sem).wait()

        def group_body(g, carry2):
            av = attr_v[pl.ds(g * 16, 16)]
            base = g * 16
            for jj in range(16):
                a = av[jj]
                for cc in range(_D // 16):
                    sl = pl.ds(cc * 16, 16)
                    rows_v[base + jj, sl] = rows_v[base + jj, sl] * a
            return carry2

        lax.fori_loop(0, _CH // 16, group_body, 0)
        pltpu.sync_copy(rows_v, acc_sh.at[dst_v], add=True)
        return carry

    lax.fori_loop(0, _NCHUNK, chunk_body, 0)
    plsc.subcore_barrier()
    pltpu.sync_copy(acc_sh.at[pl.ds(s * _RPT, _RPT)],
                    out_hbm.at[c, pl.ds(s * _RPT, _RPT)])


@functools.cache
def _sc_propagate_kernel():
    return pl.kernel(
        _sc_propagate_body,
        out_type=jax.ShapeDtypeStruct((_NC, _NPAD, _D), jnp.float32),
        mesh=plsc.VectorSubcoreMesh(core_axis_name="c", subcore_axis_name="s",
                                    num_cores=_NC, num_subcores=_NS),
        scratch_types=[
            pltpu.VMEM((_CH,), jnp.int32),
            pltpu.VMEM((_CH,), jnp.int32),
            pltpu.VMEM((_CH,), jnp.float32),
            pltpu.VMEM((_CH, _D), jnp.float32),
            pltpu.SemaphoreType.DMA,
            pltpu.VMEM_SHARED((_NPAD, _D), jnp.float32),
        ],
    )


def _sc_propagate(m, src_p, dst_p, attr_p, zero_rows):
    parts = _sc_propagate_kernel()(m, src_p, dst_p, attr_p, zero_rows)
    return parts[:, :_N, :]


_BLK = 1000


def _mm_body(h_ref, w_ref, o_ref):
    o_ref[...] = jnp.dot(h_ref[...], w_ref[...],
                         preferred_element_type=jnp.float32)


def _tc_matmul(h, w):
    return pl.pallas_call(
        _mm_body,
        grid=(_N // _BLK,),
        in_specs=[
            pl.BlockSpec((_BLK, _D), lambda i: (i, 0)),
            pl.BlockSpec((_D, _D), lambda i: (0, 0)),
        ],
        out_specs=pl.BlockSpec((_BLK, _D), lambda i: (i, 0)),
        out_shape=jax.ShapeDtypeStruct((_N, _D), jnp.float32),
    )(h, w)


def _gru_body(p0_ref, p1_ref, h_ref, wih_ref, whh_ref, bih_ref, bhh_ref,
              o_ref):
    agg = p0_ref[...] + p1_ref[...]
    h = h_ref[...]
    gi = lax.dot_general(agg, wih_ref[...], (((1,), (1,)), ((), ())),
                         preferred_element_type=jnp.float32) + bih_ref[...]
    gh = lax.dot_general(h, whh_ref[...], (((1,), (1,)), ((), ())),
                         preferred_element_type=jnp.float32) + bhh_ref[...]
    r = jax.nn.sigmoid(gi[:, :_D] + gh[:, :_D])
    z = jax.nn.sigmoid(gi[:, _D:2 * _D] + gh[:, _D:2 * _D])
    n = jnp.tanh(gi[:, 2 * _D:] + r * gh[:, 2 * _D:])
    o_ref[...] = (1.0 - z) * n + z * h


def _tc_gru(p0, p1, h, w_ih, w_hh, b_ih, b_hh):
    return pl.pallas_call(
        _gru_body,
        grid=(_N // _BLK,),
        in_specs=[
            pl.BlockSpec((_BLK, _D), lambda i: (i, 0)),
            pl.BlockSpec((_BLK, _D), lambda i: (i, 0)),
            pl.BlockSpec((_BLK, _D), lambda i: (i, 0)),
            pl.BlockSpec((3 * _D, _D), lambda i: (0, 0)),
            pl.BlockSpec((3 * _D, _D), lambda i: (0, 0)),
            pl.BlockSpec((1, 3 * _D), lambda i: (0, 0)),
            pl.BlockSpec((1, 3 * _D), lambda i: (0, 0)),
        ],
        out_specs=pl.BlockSpec((_BLK, _D), lambda i: (i, 0)),
        out_shape=jax.ShapeDtypeStruct((_N, _D), jnp.float32),
    )(p0, p1, h, w_ih, w_hh, b_ih, b_hh)


def _mlp_body(h_ref, w0_ref, b0_ref, w1_ref, b1_ref, w2_ref, b2_ref,
              ow_ref, ob_ref, emb_ref, lsm_ref):
    y = jnp.tanh(lax.dot_general(h_ref[...], w0_ref[...],
                                 (((1,), (1,)), ((), ())),
                                 preferred_element_type=jnp.float32)
                 + b0_ref[...])
    y = jnp.tanh(lax.dot_general(y, w1_ref[...], (((1,), (1,)), ((), ())),
                                 preferred_element_type=jnp.float32)
                 + b1_ref[...])
    y = jnp.tanh(lax.dot_general(y, w2_ref[...], (((1,), (1,)), ((), ())),
                                 preferred_element_type=jnp.float32)
                 + b2_ref[...])
    e = lax.dot_general(y, ow_ref[...], (((1,), (1,)), ((), ())),
                        preferred_element_type=jnp.float32) + ob_ref[...]
    emb_ref[...] = e
    shifted = e - jnp.max(e, axis=-1, keepdims=True)
    lsm_ref[...] = shifted - jnp.log(
        jnp.sum(jnp.exp(shifted), axis=-1, keepdims=True))


def _tc_mlp(h, w0, b0, w1, b1, w2, b2, ow, ob):
    return pl.pallas_call(
        _mlp_body,
        grid=(_N // _BLK,),
        in_specs=[
            pl.BlockSpec((_BLK, _D), lambda i: (i, 0)),
            pl.BlockSpec((_MLP_H, _D), lambda i: (0, 0)),
            pl.BlockSpec((1, _MLP_H), lambda i: (0, 0)),
            pl.BlockSpec((_MLP_H, _MLP_H), lambda i: (0, 0)),
            pl.BlockSpec((1, _MLP_H), lambda i: (0, 0)),
            pl.BlockSpec((_MLP_H, _MLP_H), lambda i: (0, 0)),
            pl.BlockSpec((1, _MLP_H), lambda i: (0, 0)),
            pl.BlockSpec((_CLS, _MLP_H), lambda i: (0, 0)),
            pl.BlockSpec((1, _CLS), lambda i: (0, 0)),
        ],
        out_specs=[
            pl.BlockSpec((_BLK, _CLS), lambda i: (i, 0)),
            pl.BlockSpec((_BLK, _CLS), lambda i: (i, 0)),
        ],
        out_shape=[
            jax.ShapeDtypeStruct((_N, _CLS), jnp.float32),
            jax.ShapeDtypeStruct((_N, _CLS), jnp.float32),
        ],
    )(h, w0, b0, w1, b1, w2, b2, ow, ob)


def kernel(x, edge_index, edge_attr, W, W_ih, W_hh, b_ih, b_hh,
           mlp_w0, mlp_b0, mlp_w1, mlp_b1, mlp_w2, mlp_b2, out_w, out_b):
    src = edge_index[0].astype(jnp.int32)
    dst = edge_index[1].astype(jnp.int32)
    attr = edge_attr.astype(jnp.float32)

    pad = _EPAD - _E
    pad_i = jnp.zeros((pad,), jnp.int32)
    src_p = jnp.concatenate([src, pad_i]).reshape(_NW, _NCHUNK, _CH)
    dst_p = jnp.concatenate([dst, pad_i]).reshape(_NW, _NCHUNK, _CH)
    attr_p = jnp.concatenate([attr, jnp.zeros((pad,), jnp.float32)]
                             ).reshape(_NW, _NCHUNK, _CH)
    zero_rows = jnp.zeros((_RPT, _D), jnp.float32)

    b_ih2 = b_ih.reshape(1, 3 * _D)
    b_hh2 = b_hh.reshape(1, 3 * _D)

    h = x
    m = _tc_matmul(h, W[0])
    for i in range(_LAYERS):
        parts = _sc_propagate(m, src_p, dst_p, attr_p, zero_rows)
        h = _tc_gru(parts[0], parts[1], h, W_ih, W_hh, b_ih2, b_hh2)
        if i + 1 < _LAYERS:
            m = _tc_matmul(h, W[i + 1])

    return _tc_mlp(h, mlp_w0, mlp_b0.reshape(1, _MLP_H),
                   mlp_w1, mlp_b1.reshape(1, _MLP_H),
                   mlp_w2, mlp_b2.reshape(1, _MLP_H),
                   out_w, out_b.reshape(1, _CLS))


# speedup vs baseline: 2.3981x; 1.7935x over previous
import functools

import jax
import jax.numpy as jnp
from jax import lax
from jax.experimental import pallas as pl
from jax.experimental.pallas import tpu as pltpu
from jax.experimental.pallas import tpu_sc as plsc

_N = 10000
_E = 320000
_D = 128
_LAYERS = 3
_MLP_H = 32
_CLS = 10

_NC = 2
_NS = 16
_NW = _NC * _NS
_CH = 112
_NCHUNK = 90
_EPAD = _NW * _NCHUNK * _CH
_NPAD = 10112
_RPT = _NPAD // _NS
_RB = 3
_PB = 6


def _sc_propagate_body(m_hbm, edges_hbm, zero_hbm, out_hbm,
                       pk_v, rows_v, sem0, sem1, sem2, sem_i, acc_sh):
    c = lax.axis_index("c")
    s = lax.axis_index("s")
    w = c * _NS + s
    sems = (sem0, sem1, sem2)

    pltpu.sync_copy(zero_hbm, acc_sh.at[pl.ds(s * _RPT, _RPT)])

    def idx_copy(j, slot):
        return pltpu.make_async_copy(edges_hbm.at[w, j], pk_v.at[slot], sem_i)

    def gather(j, slot, rb):
        return pltpu.make_async_copy(m_hbm.at[pk_v.at[slot, 0]],
                                     rows_v.at[rb], sems[rb])

    for j0 in range(_PB):
        idx_copy(j0, j0).start()
    for j0 in range(_RB):
        idx_copy(j0, j0).wait()
        gather(j0, j0, j0).start()
    plsc.subcore_barrier()

    def six_body(p, carry):
        for ph in range(_PB):
            j = _PB * p + ph
            rb = ph % _RB
            gather(j, ph, rb).wait()

            def group_body(g, carry2):
                av = lax.bitcast_convert_type(
                    pk_v[ph, 2, pl.ds(g * 16, 16)], jnp.float32)
                base = g * 16
                for jj in range(16):
                    a = av[jj]
                    for cc in range(_D // 16):
                        sl = pl.ds(cc * 16, 16)
                        rows_v[rb, base + jj, sl] = (
                            rows_v[rb, base + jj, sl] * a)
                return carry2

            lax.fori_loop(0, _CH // 16, group_body, 0)
            pltpu.sync_copy(rows_v.at[rb], acc_sh.at[pk_v.at[ph, 1]],
                            add=True)

            @pl.when(j + _PB < _NCHUNK)
            def _():
                idx_copy(j + _PB, ph).start()

            @pl.when(j + _RB < _NCHUNK)
            def _():
                idx_copy(j + _RB, (ph + _RB) % _PB).wait()
                gather(j + _RB, (ph + _RB) % _PB, rb).start()
        return carry

    lax.fori_loop(0, _NCHUNK // _PB, six_body, 0)
    plsc.subcore_barrier()
    pltpu.sync_copy(acc_sh.at[pl.ds(s * _RPT, _RPT)],
                    out_hbm.at[c, pl.ds(s * _RPT, _RPT)])


@functools.cache
def _sc_propagate_kernel():
    return pl.kernel(
        _sc_propagate_body,
        out_type=jax.ShapeDtypeStruct((_NC, _NPAD, _D), jnp.float32),
        mesh=plsc.VectorSubcoreMesh(core_axis_name="c", subcore_axis_name="s",
                                    num_cores=_NC, num_subcores=_NS),
        scratch_types=[
            pltpu.VMEM((_PB, 3, _CH), jnp.int32),
            pltpu.VMEM((_RB, _CH, _D), jnp.float32),
            pltpu.SemaphoreType.DMA,
            pltpu.SemaphoreType.DMA,
            pltpu.SemaphoreType.DMA,
            pltpu.SemaphoreType.DMA,
            pltpu.VMEM_SHARED((_NPAD, _D), jnp.float32),
        ],
    )


def _sc_propagate(m, edges_p, zero_rows):
    parts = _sc_propagate_kernel()(m, edges_p, zero_rows)
    return parts[:, :_N, :]


_BLK = 1000


def _mm_body(h_ref, w_ref, o_ref):
    o_ref[...] = jnp.dot(h_ref[...], w_ref[...],
                         preferred_element_type=jnp.float32)


def _tc_matmul(h, w):
    return pl.pallas_call(
        _mm_body,
        grid=(_N // _BLK,),
        in_specs=[
            pl.BlockSpec((_BLK, _D), lambda i: (i, 0)),
            pl.BlockSpec((_D, _D), lambda i: (0, 0)),
        ],
        out_specs=pl.BlockSpec((_BLK, _D), lambda i: (i, 0)),
        out_shape=jax.ShapeDtypeStruct((_N, _D), jnp.float32),
    )(h, w)


def _gru_body(p0_ref, p1_ref, h_ref, wih_ref, whh_ref, bih_ref, bhh_ref,
              o_ref):
    agg = p0_ref[...] + p1_ref[...]
    h = h_ref[...]
    gi = lax.dot_general(agg, wih_ref[...], (((1,), (1,)), ((), ())),
                         preferred_element_type=jnp.float32) + bih_ref[...]
    gh = lax.dot_general(h, whh_ref[...], (((1,), (1,)), ((), ())),
                         preferred_element_type=jnp.float32) + bhh_ref[...]
    r = jax.nn.sigmoid(gi[:, :_D] + gh[:, :_D])
    z = jax.nn.sigmoid(gi[:, _D:2 * _D] + gh[:, _D:2 * _D])
    n = jnp.tanh(gi[:, 2 * _D:] + r * gh[:, 2 * _D:])
    o_ref[...] = (1.0 - z) * n + z * h


def _tc_gru(p0, p1, h, w_ih, w_hh, b_ih, b_hh):
    return pl.pallas_call(
        _gru_body,
        grid=(_N // _BLK,),
        in_specs=[
            pl.BlockSpec((_BLK, _D), lambda i: (i, 0)),
            pl.BlockSpec((_BLK, _D), lambda i: (i, 0)),
            pl.BlockSpec((_BLK, _D), lambda i: (i, 0)),
            pl.BlockSpec((3 * _D, _D), lambda i: (0, 0)),
            pl.BlockSpec((3 * _D, _D), lambda i: (0, 0)),
            pl.BlockSpec((1, 3 * _D), lambda i: (0, 0)),
            pl.BlockSpec((1, 3 * _D), lambda i: (0, 0)),
        ],
        out_specs=pl.BlockSpec((_BLK, _D), lambda i: (i, 0)),
        out_shape=jax.ShapeDtypeStruct((_N, _D), jnp.float32),
    )(p0, p1, h, w_ih, w_hh, b_ih, b_hh)


def _mlp_body(h_ref, w0_ref, b0_ref, w1_ref, b1_ref, w2_ref, b2_ref,
              ow_ref, ob_ref, emb_ref, lsm_ref):
    y = jnp.tanh(lax.dot_general(h_ref[...], w0_ref[...],
                                 (((1,), (1,)), ((), ())),
                                 preferred_element_type=jnp.float32)
                 + b0_ref[...])
    y = jnp.tanh(lax.dot_general(y, w1_ref[...], (((1,), (1,)), ((), ())),
                                 preferred_element_type=jnp.float32)
                 + b1_ref[...])
    y = jnp.tanh(lax.dot_general(y, w2_ref[...], (((1,), (1,)), ((), ())),
                                 preferred_element_type=jnp.float32)
                 + b2_ref[...])
    e = lax.dot_general(y, ow_ref[...], (((1,), (1,)), ((), ())),
                        preferred_element_type=jnp.float32) + ob_ref[...]
    emb_ref[...] = e
    shifted = e - jnp.max(e, axis=-1, keepdims=True)
    lsm_ref[...] = shifted - jnp.log(
        jnp.sum(jnp.exp(shifted), axis=-1, keepdims=True))


def _tc_mlp(h, w0, b0, w1, b1, w2, b2, ow, ob):
    return pl.pallas_call(
        _mlp_body,
        grid=(_N // _BLK,),
        in_specs=[
            pl.BlockSpec((_BLK, _D), lambda i: (i, 0)),
            pl.BlockSpec((_MLP_H, _D), lambda i: (0, 0)),
            pl.BlockSpec((1, _MLP_H), lambda i: (0, 0)),
            pl.BlockSpec((_MLP_H, _MLP_H), lambda i: (0, 0)),
            pl.BlockSpec((1, _MLP_H), lambda i: (0, 0)),
            pl.BlockSpec((_MLP_H, _MLP_H), lambda i: (0, 0)),
            pl.BlockSpec((1, _MLP_H), lambda i: (0, 0)),
            pl.BlockSpec((_CLS, _MLP_H), lambda i: (0, 0)),
            pl.BlockSpec((1, _CLS), lambda i: (0, 0)),
        ],
        out_specs=[
            pl.BlockSpec((_BLK, _CLS), lambda i: (i, 0)),
            pl.BlockSpec((_BLK, _CLS), lambda i: (i, 0)),
        ],
        out_shape=[
            jax.ShapeDtypeStruct((_N, _CLS), jnp.float32),
            jax.ShapeDtypeStruct((_N, _CLS), jnp.float32),
        ],
    )(h, w0, b0, w1, b1, w2, b2, ow, ob)


def kernel(x, edge_index, edge_attr, W, W_ih, W_hh, b_ih, b_hh,
           mlp_w0, mlp_b0, mlp_w1, mlp_b1, mlp_w2, mlp_b2, out_w, out_b):
    src = edge_index[0].astype(jnp.int32)
    dst = edge_index[1].astype(jnp.int32)
    attr = edge_attr.astype(jnp.float32)

    pad = _EPAD - _E
    pad_i = jnp.zeros((pad,), jnp.int32)
    pad_d = _N + (jnp.arange(pad, dtype=jnp.int32) % (_NPAD - _N))
    src_p = jnp.concatenate([src, pad_i]).reshape(_NW, _NCHUNK, _CH)
    dst_p = jnp.concatenate([dst, pad_d]).reshape(_NW, _NCHUNK, _CH)
    attr_p = jnp.concatenate(
        [lax.bitcast_convert_type(attr, jnp.int32), pad_i]
    ).reshape(_NW, _NCHUNK, _CH)
    edges_p = jnp.stack([src_p, dst_p, attr_p], axis=2)
    zero_rows = jnp.zeros((_RPT, _D), jnp.float32)

    b_ih2 = b_ih.reshape(1, 3 * _D)
    b_hh2 = b_hh.reshape(1, 3 * _D)

    h = x
    m = _tc_matmul(h, W[0])
    for i in range(_LAYERS):
        parts = _sc_propagate(m, edges_p, zero_rows)
        h = _tc_gru(parts[0], parts[1], h, W_ih, W_hh, b_ih2, b_hh2)
        if i + 1 < _LAYERS:
            m = _tc_matmul(h, W[i + 1])

    return _tc_mlp(h, mlp_w0, mlp_b0.reshape(1, _MLP_H),
                   mlp_w1, mlp_b1.reshape(1, _MLP_H),
                   mlp_w2, mlp_b2.reshape(1, _MLP_H),
                   out_w, out_b.reshape(1, _CLS))
